# Initial kernel scaffold; baseline (speedup 1.0000x reference)
import jax
import jax.numpy as jnp
from jax.experimental import pallas as pl


def kernel(x, edge_index_q, edge_index_p, Wq0, bq0, Wp0, bp0, Wq1, bq1, Wp1, bp1, Wq2, bq2, Wp2, bp2):
    raise NotImplementedError("write your pallas kernel here")


# speedup vs baseline: 3.2336x; 3.2336x over previous
import functools

import jax
import jax.numpy as jnp
from jax import lax
from jax.experimental import pallas as pl
from jax.experimental.pallas import tpu as pltpu
from jax.experimental.pallas import tpu_sc as plsc

ALPHA_Q = 0.7
ALPHA_P = 0.3

NC = 2
NS = 16
B = 80
CG = 128


def _mesh():
    return plsc.VectorSubcoreMesh(core_axis_name="c", subcore_axis_name="s")


@functools.lru_cache(maxsize=None)
def _tc_degree_kernel(E, NPAD):
    EB = 4000
    NT = E // EB
    HI = NPAD // CG

    def body(dq_ref, dp_ref, oq_ref, op_ref):
        i = pl.program_id(0)

        @pl.when(i == 0)
        def _():
            oq_ref[...] = jnp.ones((HI, CG), jnp.float32)
            op_ref[...] = jnp.ones((HI, CG), jnp.float32)

        hi_io = lax.broadcasted_iota(jnp.int32, (1, HI), 1)
        lo_io = lax.broadcasted_iota(jnp.int32, (1, CG), 1)
        for ref, o in ((dq_ref, oq_ref), (dp_ref, op_ref)):
            d = ref[...]
            a = ((d >> 7) == hi_io).astype(jnp.float32)
            b = ((d & (CG - 1)) == lo_io).astype(jnp.float32)
            o[...] += lax.dot_general(a, b, (((0,), (0,)), ((), ())),
                                      preferred_element_type=jnp.float32)

    eb = pl.BlockSpec((EB, 1), lambda i: (i, 0))
    ob = pl.BlockSpec((HI, CG), lambda i: (0, 0))
    return pl.pallas_call(
        body,
        grid=(NT,),
        in_specs=[eb, eb],
        out_specs=[ob, ob],
        out_shape=[jax.ShapeDtypeStruct((HI, CG), jnp.float32),
                   jax.ShapeDtypeStruct((HI, CG), jnp.float32)],
    )


@functools.lru_cache(maxsize=None)
def _sc_conv_kernel(E, NPAD, split):
    ept = E // NS
    nblk = ept // B
    NLOC = NPAD // NC
    ACCR = NLOC + CG
    zrt = ACCR // NS
    rpt = NLOC // NS

    @functools.partial(
        pl.kernel,
        mesh=_mesh(),
        out_type=[jax.ShapeDtypeStruct((split, NPAD, CG), jnp.float32),
                  jax.ShapeDtypeStruct((split, NPAD, CG), jnp.float32)],
        scratch_types=[
            pltpu.VMEM((B,), jnp.int32),
            pltpu.VMEM((B,), jnp.int32),
            pltpu.VMEM((B,), jnp.int32),
            pltpu.VMEM((B,), jnp.int32),
            pltpu.VMEM((B, CG), jnp.float32),
            pltpu.VMEM((8, CG), jnp.float32),
            pltpu.VMEM((rpt, CG), jnp.float32),
            pltpu.VMEM_SHARED((ACCR, CG), jnp.float32),
            pltpu.SemaphoreType.DMA,
        ],
    )
    def k(g2q_hbm, srcq_hbm, dstq_hbm, g2p_hbm, srcp_hbm, dstp_hbm,
          outq_hbm, outp_hbm, srcv, gidx, dstv, locv, rows, zb, fb, acc, sem):
        c = lax.axis_index("c")
        s = lax.axis_index("s")
        lo = c * NLOC
        for i in range(8):
            for j in range(CG // 16):
                zb[i, pl.ds(j * 16, 16)] = jnp.zeros((16,), jnp.float32)

        def run_pass(g2_hbm, src_hbm, dst_hbm, out_hbm, grp):
            def zf(i, _):
                pltpu.sync_copy(zb, acc.at[pl.ds(s * zrt + i * 8, 8)])
                return 0
            lax.fori_loop(0, zrt // 8, zf, 0)
            plsc.subcore_barrier()

            def blk(i, _):
                base = s * ept + i * B
                pltpu.sync_copy(src_hbm.at[pl.ds(base, B)], srcv)
                pltpu.sync_copy(dst_hbm.at[pl.ds(base, B)], dstv)
                for j in range(B // 16):
                    sl = pl.ds(j * 16, 16)
                    if split > 1:
                        gidx[sl] = srcv[sl] * split + grp
                    else:
                        gidx[sl] = srcv[sl]
                    d = dstv[sl] - lo
                    ok = (d >= 0) & (d < NLOC)
                    locv[sl] = jnp.where(ok, d, NLOC + (dstv[sl] & (CG - 1)))
                pltpu.async_copy(g2_hbm.at[gidx], rows, sem).wait()
                pltpu.sync_copy(rows, acc.at[locv], add=True)
                return 0
            lax.fori_loop(0, nblk, blk, 0)
            plsc.subcore_barrier()

            pltpu.sync_copy(acc.at[pl.ds(s * rpt, rpt)], fb)
            pltpu.sync_copy(fb, out_hbm.at[grp, pl.ds(lo + s * rpt, rpt)])
            plsc.subcore_barrier()

        for grp in range(split):
            run_pass(g2q_hbm, srcq_hbm, dstq_hbm, outq_hbm, grp)
            run_pass(g2p_hbm, srcp_hbm, dstp_hbm, outp_hbm, grp)

    return k


def _sc_conv(gq, gp, srcq, dstq, srcp, dstp, NPAD):
    N, F = gq.shape
    split = F // CG
    k = _sc_conv_kernel(srcq.shape[0], NPAD, split)
    return k(gq.reshape(split * N, CG), srcq, dstq,
             gp.reshape(split * N, CG), srcp, dstp)


@functools.lru_cache(maxsize=None)
def _tc_prep_kernel(N, F_IN, HID, NPAD):
    def body(x_ref, wq_ref, wp_ref, degq_ref, degp_ref, gq_ref, gp_ref):
        x = x_ref[...]
        mu = jnp.mean(x, axis=0, keepdims=True)
        xc = x - mu
        var = jnp.sum(xc * xc, axis=0, keepdims=True) / (N - 1)
        xs = xc * lax.rsqrt(var)
        dq = lax.rsqrt(degq_ref[...][:N])
        dp = lax.rsqrt(degp_ref[...][:N])
        gq_ref[...] = dq * jnp.dot(xs, wq_ref[...],
                                   preferred_element_type=jnp.float32)
        gp_ref[...] = dp * jnp.dot(xs, wp_ref[...],
                                   preferred_element_type=jnp.float32)

    return pl.pallas_call(
        body,
        out_shape=[jax.ShapeDtypeStruct((N, HID), jnp.float32),
                   jax.ShapeDtypeStruct((N, HID), jnp.float32)],
    )


@functools.lru_cache(maxsize=None)
def _tc_mid_kernel(N, F, FOUT, NPAD, TN, pad_out):
    NT = N // TN
    split = F // CG

    def body(accq_ref, accp_ref, gq_ref, gp_ref, degq_ref, degp_ref,
             bq_ref, bp_ref, wq_ref, wp_ref, gqn_ref, gpn_ref):
        dq = lax.rsqrt(degq_ref[...])
        dp = lax.rsqrt(degp_ref[...])
        accq = jnp.concatenate([accq_ref[g] for g in range(split)], axis=1)
        accp = jnp.concatenate([accp_ref[g] for g in range(split)], axis=1)
        outq = dq * accq + dq * gq_ref[...] + bq_ref[...]
        outp = dp * accp + dp * gp_ref[...] + bp_ref[...]
        x = ALPHA_Q * jnp.maximum(outq, 0.0) + ALPHA_P * jnp.maximum(outp, 0.0)
        hq = jnp.dot(x, wq_ref[...], preferred_element_type=jnp.float32)
        hp = jnp.dot(x, wp_ref[...], preferred_element_type=jnp.float32)
        if pad_out:
            z = jnp.zeros((TN, CG - FOUT), jnp.float32)
            gqn_ref[...] = jnp.concatenate([dq * hq, z], axis=1)
            gpn_ref[...] = jnp.concatenate([dp * hp, z], axis=1)
        else:
            gqn_ref[...] = dq * hq
            gpn_ref[...] = dp * hp

    fo = CG if pad_out else FOUT
    acc_s = pl.BlockSpec((split, TN, CG), lambda i: (0, i, 0))
    row = pl.BlockSpec((TN, F), lambda i: (i, 0))
    col = pl.BlockSpec((TN, 1), lambda i: (i, 0))
    full_b = pl.BlockSpec((1, F), lambda i: (0, 0))
    full_w = pl.BlockSpec((F, FOUT), lambda i: (0, 0))
    out_row = pl.BlockSpec((TN, fo), lambda i: (i, 0))
    return pl.pallas_call(
        body,
        grid=(NT,),
        in_specs=[acc_s, acc_s, row, row, col, col, full_b, full_b,
                  full_w, full_w],
        out_specs=[out_row, out_row],
        out_shape=[jax.ShapeDtypeStruct((N, fo), jnp.float32),
                   jax.ShapeDtypeStruct((N, fo), jnp.float32)],
    )


@functools.lru_cache(maxsize=None)
def _tc_final_kernel(N, F, NPAD, TN):
    NT = N // TN

    def body(accq_ref, accp_ref, gq_ref, gp_ref, degq_ref, degp_ref,
             bq_ref, bp_ref, out_ref):
        dq = lax.rsqrt(degq_ref[...])
        dp = lax.rsqrt(degp_ref[...])
        accq = accq_ref[0][:, :F]
        accp = accp_ref[0][:, :F]
        gq = gq_ref[...][:, :F]
        gp = gp_ref[...][:, :F]
        outq = dq * accq + dq * gq + bq_ref[...]
        outp = dp * accp + dp * gp + bp_ref[...]
        z = ALPHA_Q * outq + ALPHA_P * outp
        m = jnp.max(z, axis=1, keepdims=True)
        zs = z - m
        lse = jnp.log(jnp.sum(jnp.exp(zs), axis=1, keepdims=True))
        out_ref[...] = zs - lse

    acc_s = pl.BlockSpec((1, TN, CG), lambda i: (0, i, 0))
    row_p = pl.BlockSpec((TN, CG), lambda i: (i, 0))
    col = pl.BlockSpec((TN, 1), lambda i: (i, 0))
    full_b = pl.BlockSpec((1, F), lambda i: (0, 0))
    out_row = pl.BlockSpec((TN, F), lambda i: (i, 0))
    return pl.pallas_call(
        body,
        grid=(NT,),
        in_specs=[acc_s, acc_s, row_p, row_p, col, col, full_b, full_b],
        out_specs=out_row,
        out_shape=jax.ShapeDtypeStruct((N, F), jnp.float32),
    )


def kernel(x, edge_index_q, edge_index_p, Wq0, bq0, Wp0, bp0,
           Wq1, bq1, Wp1, bp1, Wq2, bq2, Wp2, bp2):
    N, F_IN = x.shape
    HID = Wq0.shape[1]
    NLAB = Wq2.shape[1]
    E = edge_index_q.shape[1]
    NPAD = 10240
    TN = 1000

    srcq, dstq = edge_index_q[0], edge_index_q[1]
    srcp, dstp = edge_index_p[0], edge_index_p[1]

    degq, degp = _tc_degree_kernel(E, NPAD)(dstq.reshape(E, 1),
                                            dstp.reshape(E, 1))
    degq = degq.reshape(NPAD, 1)
    degp = degp.reshape(NPAD, 1)

    gq, gp = _tc_prep_kernel(N, F_IN, HID, NPAD)(x, Wq0, Wp0, degq, degp)

    mid = _tc_mid_kernel(N, HID, HID, NPAD, TN, False)
    last = _tc_mid_kernel(N, HID, NLAB, NPAD, TN, True)

    accq, accp = _sc_conv(gq, gp, srcq, dstq, srcp, dstp, NPAD)
    gq, gp = mid(accq, accp, gq, gp, degq, degp,
                 bq0.reshape(1, HID), bp0.reshape(1, HID), Wq1, Wp1)

    accq, accp = _sc_conv(gq, gp, srcq, dstq, srcp, dstp, NPAD)
    gq, gp = mid(accq, accp, gq, gp, degq, degp,
                 bq1.reshape(1, HID), bp1.reshape(1, HID), Wq1, Wp1)

    accq, accp = _sc_conv(gq, gp, srcq, dstq, srcp, dstp, NPAD)
    gq, gp = last(accq, accp, gq, gp, degq, degp,
                  bq1.reshape(1, HID), bp1.reshape(1, HID), Wq2, Wp2)

    accq, accp = _sc_conv(gq, gp, srcq, dstq, srcp, dstp, NPAD)
    out = _tc_final_kernel(N, NLAB, NPAD, TN)(
        accq, accp, gq, gp, degq, degp,
        bq2.reshape(1, NLAB), bp2.reshape(1, NLAB))
    return out


# speedup vs baseline: 6.5683x; 2.0313x over previous
import functools

import jax
import jax.numpy as jnp
from jax import lax
from jax.experimental import pallas as pl
from jax.experimental.pallas import tpu as pltpu
from jax.experimental.pallas import tpu_sc as plsc

ALPHA_Q = 0.7
ALPHA_P = 0.3

NC = 2
NS = 16
B = 80
CG = 128


def _mesh():
    return plsc.VectorSubcoreMesh(core_axis_name="c", subcore_axis_name="s")


@functools.lru_cache(maxsize=None)
def _tc_degree_kernel(E, NPAD):
    EB = 4000
    NT = E // EB
    HI = NPAD // CG

    def body(dq_ref, dp_ref, oq_ref, op_ref):
        i = pl.program_id(0)

        @pl.when(i == 0)
        def _():
            oq_ref[...] = jnp.ones((HI, CG), jnp.float32)
            op_ref[...] = jnp.ones((HI, CG), jnp.float32)

        hi_io = lax.broadcasted_iota(jnp.int32, (1, HI), 1)
        lo_io = lax.broadcasted_iota(jnp.int32, (1, CG), 1)
        for ref, o in ((dq_ref, oq_ref), (dp_ref, op_ref)):
            d = ref[...]
            a = ((d >> 7) == hi_io).astype(jnp.float32)
            b = ((d & (CG - 1)) == lo_io).astype(jnp.float32)
            o[...] += lax.dot_general(a, b, (((0,), (0,)), ((), ())),
                                      preferred_element_type=jnp.float32)

    eb = pl.BlockSpec((EB, 1), lambda i: (i, 0))
    ob = pl.BlockSpec((HI, CG), lambda i: (0, 0))
    return pl.pallas_call(
        body,
        grid=(NT,),
        in_specs=[eb, eb],
        out_specs=[ob, ob],
        out_shape=[jax.ShapeDtypeStruct((HI, CG), jnp.float32),
                   jax.ShapeDtypeStruct((HI, CG), jnp.float32)],
    )


@functools.lru_cache(maxsize=None)
def _sc_conv_kernel(E, NPAD, split):
    ept = E // NS
    BB = 128
    nfull = ept // BB
    tail = ept - nfull * BB
    assert tail % 8 == 0 and nfull % 2 == 0
    tsz = max(tail, 8)
    NLOC = NPAD // NC
    ACCR = NLOC + CG
    zrt = ACCR // NS
    rpt = NLOC // NS

    @functools.partial(
        pl.kernel,
        mesh=_mesh(),
        out_type=[jax.ShapeDtypeStruct((split, NPAD, CG), jnp.float32),
                  jax.ShapeDtypeStruct((split, NPAD, CG), jnp.float32)],
        scratch_types=[
            pltpu.VMEM((BB,), jnp.int32),
            pltpu.VMEM((BB,), jnp.int32),
            pltpu.VMEM((BB,), jnp.int32),
            pltpu.VMEM((BB,), jnp.int32),
            pltpu.VMEM((BB, CG), jnp.float32),
            pltpu.VMEM((BB, CG), jnp.float32),
            pltpu.VMEM((tsz,), jnp.int32),
            pltpu.VMEM((tsz,), jnp.int32),
            pltpu.VMEM((tsz, CG), jnp.float32),
            pltpu.VMEM((8, CG), jnp.float32),
            pltpu.VMEM((rpt, CG), jnp.float32),
            pltpu.VMEM_SHARED((ACCR, CG), jnp.float32),
            pltpu.SemaphoreType.DMA,
            pltpu.SemaphoreType.DMA,
        ],
    )
    def k(g2q_hbm, srcq_hbm, dstq_hbm, g2p_hbm, srcp_hbm, dstp_hbm,
          outq_hbm, outp_hbm, ev0, ev1, dv0, dv1, rows0, rows1,
          evt, dvt, rowst, zb, fb, acc, sem0, sem1):
        c = lax.axis_index("c")
        s = lax.axis_index("s")
        lo = c * NLOC
        ev = (ev0, ev1)
        dv = (dv0, dv1)
        rows = (rows0, rows1)
        sem = (sem0, sem1)
        for i in range(8):
            for j in range(CG // 16):
                zb[i, pl.ds(j * 16, 16)] = jnp.zeros((16,), jnp.float32)

        def run_pass(g2_hbm, src_hbm, dst_hbm, out_hbm, grp):
            def zf(i, _):
                pltpu.sync_copy(zb, acc.at[pl.ds(s * zrt + i * 8, 8)])
                return 0
            lax.fori_loop(0, zrt // 8, zf, 0)
            plsc.subcore_barrier()

            def load(i, e_v, d_v, n):
                base = s * ept + i * BB
                pltpu.sync_copy(src_hbm.at[pl.ds(base, n)], e_v)
                pltpu.sync_copy(dst_hbm.at[pl.ds(base, n)], d_v)
                for j in range(n // 16):
                    sl = pl.ds(j * 16, 16)
                    if split > 1:
                        e_v[sl] = e_v[sl] * split + grp
                    d = d_v[sl] - lo
                    ok = (d >= 0) & (d < NLOC)
                    d_v[sl] = jnp.where(ok, d, NLOC + (d_v[sl] & (CG - 1)))

            def gstart(p):
                return pltpu.async_copy(g2_hbm.at[ev[p]], rows[p], sem[p])

            def gwait(p):
                pltpu.make_async_copy(g2_hbm.at[ev[p]], rows[p], sem[p]).wait()

            def scat(p):
                pltpu.sync_copy(rows[p], acc.at[dv[p]], add=True)

            load(0, ev[0], dv[0], BB)
            gstart(0)

            def pair(k2, _):
                i = 2 * k2
                load(i + 1, ev[1], dv[1], BB)
                gwait(0)
                gstart(1)
                scat(0)
                load(i + 2, ev[0], dv[0], BB)
                gwait(1)
                gstart(0)
                scat(1)
                return 0
            lax.fori_loop(0, (nfull - 2) // 2, pair, 0)

            load(nfull - 1, ev[1], dv[1], BB)
            gwait(0)
            gstart(1)
            scat(0)
            if tail:
                load(nfull, evt, dvt, tail)
                tc = pltpu.async_copy(g2_hbm.at[evt], rowst, sem[0])
            gwait(1)
            scat(1)
            if tail:
                tc.wait()
                pltpu.sync_copy(rowst, acc.at[dvt], add=True)
            plsc.subcore_barrier()

            pltpu.sync_copy(acc.at[pl.ds(s * rpt, rpt)], fb)
            pltpu.sync_copy(fb, out_hbm.at[grp, pl.ds(lo + s * rpt, rpt)])
            plsc.subcore_barrier()

        for grp in range(split):
            run_pass(g2q_hbm, srcq_hbm, dstq_hbm, outq_hbm, grp)
            run_pass(g2p_hbm, srcp_hbm, dstp_hbm, outp_hbm, grp)

    return k


def _sc_conv(gq, gp, srcq, dstq, srcp, dstp, NPAD):
    N, F = gq.shape
    split = F // CG
    k = _sc_conv_kernel(srcq.shape[0], NPAD, split)
    return k(gq.reshape(split * N, CG), srcq, dstq,
             gp.reshape(split * N, CG), srcp, dstp)


@functools.lru_cache(maxsize=None)
def _tc_prep_kernel(N, F_IN, HID, NPAD):
    def body(x_ref, wq_ref, wp_ref, degq_ref, degp_ref, gq_ref, gp_ref):
        x = x_ref[...]
        mu = jnp.mean(x, axis=0, keepdims=True)
        xc = x - mu
        var = jnp.sum(xc * xc, axis=0, keepdims=True) / (N - 1)
        xs = xc * lax.rsqrt(var)
        dq = lax.rsqrt(degq_ref[...][:N])
        dp = lax.rsqrt(degp_ref[...][:N])
        gq_ref[...] = dq * jnp.dot(xs, wq_ref[...],
                                   preferred_element_type=jnp.float32)
        gp_ref[...] = dp * jnp.dot(xs, wp_ref[...],
                                   preferred_element_type=jnp.float32)

    return pl.pallas_call(
        body,
        out_shape=[jax.ShapeDtypeStruct((N, HID), jnp.float32),
                   jax.ShapeDtypeStruct((N, HID), jnp.float32)],
    )


@functools.lru_cache(maxsize=None)
def _tc_mid_kernel(N, F, FOUT, NPAD, TN, pad_out):
    NT = N // TN
    split = F // CG

    def body(accq_ref, accp_ref, gq_ref, gp_ref, degq_ref, degp_ref,
             bq_ref, bp_ref, wq_ref, wp_ref, gqn_ref, gpn_ref):
        dq = lax.rsqrt(degq_ref[...])
        dp = lax.rsqrt(degp_ref[...])
        accq = jnp.concatenate([accq_ref[g] for g in range(split)], axis=1)
        accp = jnp.concatenate([accp_ref[g] for g in range(split)], axis=1)
        outq = dq * accq + dq * gq_ref[...] + bq_ref[...]
        outp = dp * accp + dp * gp_ref[...] + bp_ref[...]
        x = ALPHA_Q * jnp.maximum(outq, 0.0) + ALPHA_P * jnp.maximum(outp, 0.0)
        hq = jnp.dot(x, wq_ref[...], preferred_element_type=jnp.float32)
        hp = jnp.dot(x, wp_ref[...], preferred_element_type=jnp.float32)
        if pad_out:
            z = jnp.zeros((TN, CG - FOUT), jnp.float32)
            gqn_ref[...] = jnp.concatenate([dq * hq, z], axis=1)
            gpn_ref[...] = jnp.concatenate([dp * hp, z], axis=1)
        else:
            gqn_ref[...] = dq * hq
            gpn_ref[...] = dp * hp

    fo = CG if pad_out else FOUT
    acc_s = pl.BlockSpec((split, TN, CG), lambda i: (0, i, 0))
    row = pl.BlockSpec((TN, F), lambda i: (i, 0))
    col = pl.BlockSpec((TN, 1), lambda i: (i, 0))
    full_b = pl.BlockSpec((1, F), lambda i: (0, 0))
    full_w = pl.BlockSpec((F, FOUT), lambda i: (0, 0))
    out_row = pl.BlockSpec((TN, fo), lambda i: (i, 0))
    return pl.pallas_call(
        body,
        grid=(NT,),
        in_specs=[acc_s, acc_s, row, row, col, col, full_b, full_b,
                  full_w, full_w],
        out_specs=[out_row, out_row],
        out_shape=[jax.ShapeDtypeStruct((N, fo), jnp.float32),
                   jax.ShapeDtypeStruct((N, fo), jnp.float32)],
    )


@functools.lru_cache(maxsize=None)
def _tc_final_kernel(N, F, NPAD, TN):
    NT = N // TN

    def body(accq_ref, accp_ref, gq_ref, gp_ref, degq_ref, degp_ref,
             bq_ref, bp_ref, out_ref):
        dq = lax.rsqrt(degq_ref[...])
        dp = lax.rsqrt(degp_ref[...])
        accq = accq_ref[0][:, :F]
        accp = accp_ref[0][:, :F]
        gq = gq_ref[...][:, :F]
        gp = gp_ref[...][:, :F]
        outq = dq * accq + dq * gq + bq_ref[...]
        outp = dp * accp + dp * gp + bp_ref[...]
        z = ALPHA_Q * outq + ALPHA_P * outp
        m = jnp.max(z, axis=1, keepdims=True)
        zs = z - m
        lse = jnp.log(jnp.sum(jnp.exp(zs), axis=1, keepdims=True))
        out_ref[...] = zs - lse

    acc_s = pl.BlockSpec((1, TN, CG), lambda i: (0, i, 0))
    row_p = pl.BlockSpec((TN, CG), lambda i: (i, 0))
    col = pl.BlockSpec((TN, 1), lambda i: (i, 0))
    full_b = pl.BlockSpec((1, F), lambda i: (0, 0))
    out_row = pl.BlockSpec((TN, F), lambda i: (i, 0))
    return pl.pallas_call(
        body,
        grid=(NT,),
        in_specs=[acc_s, acc_s, row_p, row_p, col, col, full_b, full_b],
        out_specs=out_row,
        out_shape=jax.ShapeDtypeStruct((N, F), jnp.float32),
    )


def kernel(x, edge_index_q, edge_index_p, Wq0, bq0, Wp0, bp0,
           Wq1, bq1, Wp1, bp1, Wq2, bq2, Wp2, bp2):
    N, F_IN = x.shape
    HID = Wq0.shape[1]
    NLAB = Wq2.shape[1]
    E = edge_index_q.shape[1]
    NPAD = 10240
    TN = 1000

    srcq, dstq = edge_index_q[0], edge_index_q[1]
    srcp, dstp = edge_index_p[0], edge_index_p[1]

    degq, degp = _tc_degree_kernel(E, NPAD)(dstq.reshape(E, 1),
                                            dstp.reshape(E, 1))
    degq = degq.reshape(NPAD, 1)
    degp = degp.reshape(NPAD, 1)

    gq, gp = _tc_prep_kernel(N, F_IN, HID, NPAD)(x, Wq0, Wp0, degq, degp)

    mid = _tc_mid_kernel(N, HID, HID, NPAD, TN, False)
    last = _tc_mid_kernel(N, HID, NLAB, NPAD, TN, True)

    accq, accp = _sc_conv(gq, gp, srcq, dstq, srcp, dstp, NPAD)
    gq, gp = mid(accq, accp, gq, gp, degq, degp,
                 bq0.reshape(1, HID), bp0.reshape(1, HID), Wq1, Wp1)

    accq, accp = _sc_conv(gq, gp, srcq, dstq, srcp, dstp, NPAD)
    gq, gp = mid(accq, accp, gq, gp, degq, degp,
                 bq1.reshape(1, HID), bp1.reshape(1, HID), Wq1, Wp1)

    accq, accp = _sc_conv(gq, gp, srcq, dstq, srcp, dstp, NPAD)
    gq, gp = last(accq, accp, gq, gp, degq, degp,
                  bq1.reshape(1, HID), bp1.reshape(1, HID), Wq2, Wp2)

    accq, accp = _sc_conv(gq, gp, srcq, dstq, srcp, dstp, NPAD)
    out = _tc_final_kernel(N, NLAB, NPAD, TN)(
        accq, accp, gq, gp, degq, degp,
        bq2.reshape(1, NLAB), bp2.reshape(1, NLAB))
    return out


# speedup vs baseline: 8.7578x; 1.3333x over previous
import functools

import jax
import jax.numpy as jnp
from jax import lax
from jax.experimental import pallas as pl
from jax.experimental.pallas import tpu as pltpu
from jax.experimental.pallas import tpu_sc as plsc

ALPHA_Q = 0.7
ALPHA_P = 0.3

NC = 2
NS = 16
B = 80
CG = 128


def _mesh():
    return plsc.VectorSubcoreMesh(core_axis_name="c", subcore_axis_name="s")


@functools.lru_cache(maxsize=None)
def _tc_degree_kernel(E, NPAD):
    EB = 4000
    NT = E // EB
    HI = NPAD // CG

    def body(dq_ref, dp_ref, oq_ref, op_ref):
        i = pl.program_id(0)

        @pl.when(i == 0)
        def _():
            oq_ref[...] = jnp.ones((HI, CG), jnp.float32)
            op_ref[...] = jnp.ones((HI, CG), jnp.float32)

        hi_io = lax.broadcasted_iota(jnp.int32, (1, HI), 1)
        lo_io = lax.broadcasted_iota(jnp.int32, (1, CG), 1)
        for ref, o in ((dq_ref, oq_ref), (dp_ref, op_ref)):
            d = ref[...]
            a = ((d >> 7) == hi_io).astype(jnp.float32)
            b = ((d & (CG - 1)) == lo_io).astype(jnp.float32)
            o[...] += lax.dot_general(a, b, (((0,), (0,)), ((), ())),
                                      preferred_element_type=jnp.float32)

    eb = pl.BlockSpec((EB, 1), lambda i: (i, 0))
    ob = pl.BlockSpec((HI, CG), lambda i: (0, 0))
    return pl.pallas_call(
        body,
        grid=(NT,),
        in_specs=[eb, eb],
        out_specs=[ob, ob],
        out_shape=[jax.ShapeDtypeStruct((HI, CG), jnp.float32),
                   jax.ShapeDtypeStruct((HI, CG), jnp.float32)],
    )


@functools.lru_cache(maxsize=None)
def _sc_conv_kernel(E, NPAD, split):
    BB = 128
    SBB = 12
    SBE = SBB * BB
    R = 4
    L = 2
    ept = E // NS
    NSB = ept // SBE
    tail = ept - NSB * SBE
    assert tail % 8 == 0 and NSB % 2 == 1 and SBB % R == 0
    tsz = max(tail, 8)
    NLOC = NPAD // NC
    ACCR = NLOC + CG
    zrt = ACCR // NS
    rpt = NLOC // NS
    FB = 80
    assert rpt % FB == 0

    @functools.partial(
        pl.kernel,
        mesh=_mesh(),
        out_type=[jax.ShapeDtypeStruct((split, NPAD, CG), jnp.float32),
                  jax.ShapeDtypeStruct((split, NPAD, CG), jnp.float32)],
        scratch_types=(
            [pltpu.VMEM((SBE,), jnp.int32),
             pltpu.VMEM((SBE,), jnp.int32),
             pltpu.VMEM((SBE,), jnp.int32),
             pltpu.VMEM((SBB, BB), jnp.int32),
             pltpu.VMEM((SBB, BB), jnp.int32),
             pltpu.VMEM((tsz,), jnp.int32)] +
            [pltpu.VMEM((BB, CG), jnp.float32) for _ in range(R)] +
            [pltpu.VMEM((8, CG), jnp.float32),
             pltpu.VMEM((FB, CG), jnp.float32),
             pltpu.VMEM_SHARED((ACCR, CG), jnp.float32)] +
            [pltpu.SemaphoreType.DMA for _ in range(2 * R + 1)]
        ),
    )
    def k(g2q_hbm, srcq_hbm, dstq_hbm, g2p_hbm, srcp_hbm, dstp_hbm,
          outq_hbm, outp_hbm, ssb0, ssb1, dtmp, dloc0, dloc1, dvt,
          r0, r1, r2, r3, zb, fb, acc,
          g0, g1, g2s, g3, s0, s1, s2, s3, isem):
        c = lax.axis_index("c")
        s = lax.axis_index("s")
        lo = c * NLOC
        ssb = (ssb0, ssb1)
        dloc = (dloc0, dloc1)
        rows = (r0, r1, r2, r3)
        gsem = (g0, g1, g2s, g3)
        ssem = (s0, s1, s2, s3)
        for i in range(8):
            for j in range(CG // 16):
                zb[i, pl.ds(j * 16, 16)] = jnp.zeros((16,), jnp.float32)

        def run_set(g2_hbm, src_hbm, dst_hbm, out_hbm):
            def run_pass(grp, _):
                def zf(i, _):
                    pltpu.sync_copy(zb, acc.at[pl.ds(s * zrt + i * 8, 8)])
                    return 0
                lax.fori_loop(0, zrt // 8, zf, 0)
                plsc.subcore_barrier()

                def load_start(si, u):
                    base = s * ept + si * SBE
                    pltpu.async_copy(src_hbm.at[pl.ds(base, SBE)], ssb[u], isem)
                    pltpu.async_copy(dst_hbm.at[pl.ds(base, SBE)], dtmp, isem)

                def load_finish(si, u):
                    base = s * ept + si * SBE
                    pltpu.make_async_copy(
                        src_hbm.at[pl.ds(base, SBE)], ssb[u], isem).wait()
                    pltpu.make_async_copy(
                        dst_hbm.at[pl.ds(base, SBE)], dtmp, isem).wait()

                    def tf(r, _):
                        for kk in range(BB // 16):
                            sl = pl.ds(r * BB + kk * 16, 16)
                            if split > 1:
                                ssb[u][sl] = ssb[u][sl] * split + grp
                            d = dtmp[sl] - lo
                            ok = (d >= 0) & (d < NLOC)
                            dloc[u][r, pl.ds(kk * 16, 16)] = jnp.where(
                                ok, d, NLOC + (dtmp[sl] & (CG - 1)))
                        return 0
                    lax.fori_loop(0, SBB, tf, 0)

                def gidx(u, jj):
                    return ssb[u].at[pl.ds(jj * BB, BB)]

                def gstart(u, jj, p):
                    pltpu.async_copy(g2_hbm.at[gidx(u, jj)], rows[p], gsem[p])

                def gwait(u, jj, p):
                    pltpu.make_async_copy(
                        g2_hbm.at[gidx(u, jj)], rows[p], gsem[p]).wait()

                def sstart(u, jj, p):
                    pltpu.async_copy(rows[p], acc.at[dloc[u].at[jj]],
                                     ssem[p], add=True)

                def swait(p):
                    pltpu.make_async_copy(rows[p], acc.at[dloc[0].at[0]],
                                          ssem[p]).wait()

                def sb_steps(si, u, first):
                    for jj in range(SBB):
                        p = jj % R
                        ph = (jj + L) % R
                        gwait(u, jj, p)
                        sstart(u, jj, p)
                        if not (first and jj < L):
                            swait(ph)
                        if jj + L < SBB:
                            gstart(u, jj + L, ph)
                        else:
                            @pl.when(si < NSB - 1)
                            def _():
                                gstart(u ^ 1, jj + L - SBB, ph)
                        if jj == 0:
                            @pl.when(si < NSB - 1)
                            def _():
                                load_start(si + 1, u ^ 1)
                        if jj == L + 1:
                            @pl.when(si < NSB - 1)
                            def _():
                                load_finish(si + 1, u ^ 1)

                load_start(0, 0)
                load_finish(0, 0)
                for jj in range(L):
                    gstart(0, jj, jj % R)
                sb_steps(0, 0, True)

                def pairfn(k2, _):
                    si = 1 + 2 * k2
                    sb_steps(si, 1, False)
                    sb_steps(si + 1, 0, False)
                    return 0
                lax.fori_loop(0, (NSB - 1) // 2, pairfn, 0)

                for jj in range(SBB - L, SBB):
                    swait(jj % R)

                if tail:
                    base = s * ept + NSB * SBE
                    pltpu.sync_copy(src_hbm.at[pl.ds(base, tail)],
                                    ssb[0].at[pl.ds(0, tail)])
                    pltpu.sync_copy(dst_hbm.at[pl.ds(base, tail)],
                                    dtmp.at[pl.ds(0, tail)])
                    for kk in range(tail // 16):
                        sl = pl.ds(kk * 16, 16)
                        if split > 1:
                            ssb[0][sl] = ssb[0][sl] * split + grp
                        d = dtmp[sl] - lo
                        ok = (d >= 0) & (d < NLOC)
                        dvt[sl] = jnp.where(ok, d, NLOC + (dtmp[sl] & (CG - 1)))
                    pltpu.async_copy(
                        g2_hbm.at[ssb[0].at[pl.ds(0, tail)]],
                        rows[0].at[pl.ds(0, tail)], gsem[0]).wait()
                    pltpu.sync_copy(rows[0].at[pl.ds(0, tail)],
                                    acc.at[dvt], add=True)
                plsc.subcore_barrier()

                for t in range(rpt // FB):
                    pltpu.sync_copy(acc.at[pl.ds(s * rpt + t * FB, FB)], fb)
                    pltpu.sync_copy(
                        fb, out_hbm.at[grp, pl.ds(lo + s * rpt + t * FB, FB)])
                plsc.subcore_barrier()
                return 0
            lax.fori_loop(0, split, run_pass, 0)

        run_set(g2q_hbm, srcq_hbm, dstq_hbm, outq_hbm)
        run_set(g2p_hbm, srcp_hbm, dstp_hbm, outp_hbm)

    return k


def _sc_conv(gq, gp, srcq, dstq, srcp, dstp, NPAD):
    N, F = gq.shape
    split = F // CG
    k = _sc_conv_kernel(srcq.shape[0], NPAD, split)
    return k(gq.reshape(split * N, CG), srcq, dstq,
             gp.reshape(split * N, CG), srcp, dstp)


@functools.lru_cache(maxsize=None)
def _tc_prep_kernel(N, F_IN, HID, NPAD):
    def body(x_ref, wq_ref, wp_ref, degq_ref, degp_ref, gq_ref, gp_ref):
        x = x_ref[...]
        mu = jnp.mean(x, axis=0, keepdims=True)
        xc = x - mu
        var = jnp.sum(xc * xc, axis=0, keepdims=True) / (N - 1)
        xs = xc * lax.rsqrt(var)
        dq = lax.rsqrt(degq_ref[...][:N])
        dp = lax.rsqrt(degp_ref[...][:N])
        gq_ref[...] = dq * jnp.dot(xs, wq_ref[...],
                                   preferred_element_type=jnp.float32)
        gp_ref[...] = dp * jnp.dot(xs, wp_ref[...],
                                   preferred_element_type=jnp.float32)

    return pl.pallas_call(
        body,
        out_shape=[jax.ShapeDtypeStruct((N, HID), jnp.float32),
                   jax.ShapeDtypeStruct((N, HID), jnp.float32)],
    )


@functools.lru_cache(maxsize=None)
def _tc_mid_kernel(N, F, FOUT, NPAD, TN, pad_out):
    NT = N // TN
    split = F // CG

    def body(accq_ref, accp_ref, gq_ref, gp_ref, degq_ref, degp_ref,
             bq_ref, bp_ref, wq_ref, wp_ref, gqn_ref, gpn_ref):
        dq = lax.rsqrt(degq_ref[...])
        dp = lax.rsqrt(degp_ref[...])
        accq = jnp.concatenate([accq_ref[g] for g in range(split)], axis=1)
        accp = jnp.concatenate([accp_ref[g] for g in range(split)], axis=1)
        outq = dq * accq + dq * gq_ref[...] + bq_ref[...]
        outp = dp * accp + dp * gp_ref[...] + bp_ref[...]
        x = ALPHA_Q * jnp.maximum(outq, 0.0) + ALPHA_P * jnp.maximum(outp, 0.0)
        hq = jnp.dot(x, wq_ref[...], preferred_element_type=jnp.float32)
        hp = jnp.dot(x, wp_ref[...], preferred_element_type=jnp.float32)
        if pad_out:
            z = jnp.zeros((TN, CG - FOUT), jnp.float32)
            gqn_ref[...] = jnp.concatenate([dq * hq, z], axis=1)
            gpn_ref[...] = jnp.concatenate([dp * hp, z], axis=1)
        else:
            gqn_ref[...] = dq * hq
            gpn_ref[...] = dp * hp

    fo = CG if pad_out else FOUT
    acc_s = pl.BlockSpec((split, TN, CG), lambda i: (0, i, 0))
    row = pl.BlockSpec((TN, F), lambda i: (i, 0))
    col = pl.BlockSpec((TN, 1), lambda i: (i, 0))
    full_b = pl.BlockSpec((1, F), lambda i: (0, 0))
    full_w = pl.BlockSpec((F, FOUT), lambda i: (0, 0))
    out_row = pl.BlockSpec((TN, fo), lambda i: (i, 0))
    return pl.pallas_call(
        body,
        grid=(NT,),
        in_specs=[acc_s, acc_s, row, row, col, col, full_b, full_b,
                  full_w, full_w],
        out_specs=[out_row, out_row],
        out_shape=[jax.ShapeDtypeStruct((N, fo), jnp.float32),
                   jax.ShapeDtypeStruct((N, fo), jnp.float32)],
    )


@functools.lru_cache(maxsize=None)
def _tc_final_kernel(N, F, NPAD, TN):
    NT = N // TN

    def body(accq_ref, accp_ref, gq_ref, gp_ref, degq_ref, degp_ref,
             bq_ref, bp_ref, out_ref):
        dq = lax.rsqrt(degq_ref[...])
        dp = lax.rsqrt(degp_ref[...])
        accq = accq_ref[0][:, :F]
        accp = accp_ref[0][:, :F]
        gq = gq_ref[...][:, :F]
        gp = gp_ref[...][:, :F]
        outq = dq * accq + dq * gq + bq_ref[...]
        outp = dp * accp + dp * gp + bp_ref[...]
        z = ALPHA_Q * outq + ALPHA_P * outp
        m = jnp.max(z, axis=1, keepdims=True)
        zs = z - m
        lse = jnp.log(jnp.sum(jnp.exp(zs), axis=1, keepdims=True))
        out_ref[...] = zs - lse

    acc_s = pl.BlockSpec((1, TN, CG), lambda i: (0, i, 0))
    row_p = pl.BlockSpec((TN, CG), lambda i: (i, 0))
    col = pl.BlockSpec((TN, 1), lambda i: (i, 0))
    full_b = pl.BlockSpec((1, F), lambda i: (0, 0))
    out_row = pl.BlockSpec((TN, F), lambda i: (i, 0))
    return pl.pallas_call(
        body,
        grid=(NT,),
        in_specs=[acc_s, acc_s, row_p, row_p, col, col, full_b, full_b],
        out_specs=out_row,
        out_shape=jax.ShapeDtypeStruct((N, F), jnp.float32),
    )


def kernel(x, edge_index_q, edge_index_p, Wq0, bq0, Wp0, bp0,
           Wq1, bq1, Wp1, bp1, Wq2, bq2, Wp2, bp2):
    N, F_IN = x.shape
    HID = Wq0.shape[1]
    NLAB = Wq2.shape[1]
    E = edge_index_q.shape[1]
    NPAD = 10240
    TN = 1000

    srcq, dstq = edge_index_q[0], edge_index_q[1]
    srcp, dstp = edge_index_p[0], edge_index_p[1]

    degq, degp = _tc_degree_kernel(E, NPAD)(dstq.reshape(E, 1),
                                            dstp.reshape(E, 1))
    degq = degq.reshape(NPAD, 1)
    degp = degp.reshape(NPAD, 1)

    gq, gp = _tc_prep_kernel(N, F_IN, HID, NPAD)(x, Wq0, Wp0, degq, degp)

    mid = _tc_mid_kernel(N, HID, HID, NPAD, TN, False)
    last = _tc_mid_kernel(N, HID, NLAB, NPAD, TN, True)

    accq, accp = _sc_conv(gq, gp, srcq, dstq, srcp, dstp, NPAD)
    gq, gp = mid(accq, accp, gq, gp, degq, degp,
                 bq0.reshape(1, HID), bp0.reshape(1, HID), Wq1, Wp1)

    accq, accp = _sc_conv(gq, gp, srcq, dstq, srcp, dstp, NPAD)
    gq, gp = mid(accq, accp, gq, gp, degq, degp,
                 bq1.reshape(1, HID), bp1.reshape(1, HID), Wq1, Wp1)

    accq, accp = _sc_conv(gq, gp, srcq, dstq, srcp, dstp, NPAD)
    gq, gp = last(accq, accp, gq, gp, degq, degp,
                  bq1.reshape(1, HID), bp1.reshape(1, HID), Wq2, Wp2)

    accq, accp = _sc_conv(gq, gp, srcq, dstq, srcp, dstp, NPAD)
    out = _tc_final_kernel(N, NLAB, NPAD, TN)(
        accq, accp, gq, gp, degq, degp,
        bq2.reshape(1, NLAB), bp2.reshape(1, NLAB))
    return out


# speedup vs baseline: 8.8081x; 1.0057x over previous
import functools

import jax
import jax.numpy as jnp
from jax import lax
from jax.experimental import pallas as pl
from jax.experimental.pallas import tpu as pltpu
from jax.experimental.pallas import tpu_sc as plsc

ALPHA_Q = 0.7
ALPHA_P = 0.3

NC = 2
NS = 16
B = 80
CG = 128


def _mesh():
    return plsc.VectorSubcoreMesh(core_axis_name="c", subcore_axis_name="s")


@functools.lru_cache(maxsize=None)
def _tc_degree_kernel(E, NPAD):
    EB = 4000
    NT = E // EB
    HI = NPAD // CG

    def body(dq_ref, dp_ref, oq_ref, op_ref):
        i = pl.program_id(0)

        @pl.when(i == 0)
        def _():
            oq_ref[...] = jnp.ones((HI, CG), jnp.float32)
            op_ref[...] = jnp.ones((HI, CG), jnp.float32)

        hi_io = lax.broadcasted_iota(jnp.int32, (1, HI), 1)
        lo_io = lax.broadcasted_iota(jnp.int32, (1, CG), 1)
        for ref, o in ((dq_ref, oq_ref), (dp_ref, op_ref)):
            d = ref[...]
            a = ((d >> 7) == hi_io).astype(jnp.float32)
            b = ((d & (CG - 1)) == lo_io).astype(jnp.float32)
            o[...] += lax.dot_general(a, b, (((0,), (0,)), ((), ())),
                                      preferred_element_type=jnp.float32)

    eb = pl.BlockSpec((EB, 1), lambda i: (i, 0))
    ob = pl.BlockSpec((HI, CG), lambda i: (0, 0))
    return pl.pallas_call(
        body,
        grid=(NT,),
        in_specs=[eb, eb],
        out_specs=[ob, ob],
        out_shape=[jax.ShapeDtypeStruct((HI, CG), jnp.float32),
                   jax.ShapeDtypeStruct((HI, CG), jnp.float32)],
    )


@functools.lru_cache(maxsize=None)
def _sc_conv_kernel(E, NPAD, split):
    BB = 128
    SBB = 12
    SBE = SBB * BB
    R = 4
    L = 2
    ept = E // NS
    NSB = ept // SBE
    tail = ept - NSB * SBE
    assert tail % 8 == 0 and NSB % 2 == 1 and SBB % R == 0
    tsz = max(tail, 8)
    NLOC = NPAD // NC
    ACCR = NLOC + CG
    zrt = ACCR // NS
    rpt = NLOC // NS
    FB = 80
    assert rpt % FB == 0

    @functools.partial(
        pl.kernel,
        mesh=_mesh(),
        out_type=[jax.ShapeDtypeStruct((split, NPAD, CG), jnp.float32),
                  jax.ShapeDtypeStruct((split, NPAD, CG), jnp.float32)],
        scratch_types=(
            [pltpu.VMEM((SBE,), jnp.int32),
             pltpu.VMEM((SBE,), jnp.int32),
             pltpu.VMEM((SBE,), jnp.int32),
             pltpu.VMEM((SBB, BB), jnp.int32),
             pltpu.VMEM((SBB, BB), jnp.int32),
             pltpu.VMEM((tsz,), jnp.int32)] +
            [pltpu.VMEM((BB, CG), jnp.float32) for _ in range(R)] +
            [pltpu.VMEM((FB, CG), jnp.float32),
             pltpu.VMEM_SHARED((ACCR, CG), jnp.float32)] +
            [pltpu.SemaphoreType.DMA for _ in range(2 * R + 1)]
        ),
    )
    def k(g2q_hbm, srcq_hbm, dstq_hbm, g2p_hbm, srcp_hbm, dstp_hbm,
          outq_hbm, outp_hbm, ssb0, ssb1, dtmp, dloc0, dloc1, dvt,
          r0, r1, r2, r3, fb, acc,
          g0, g1, g2s, g3, s0, s1, s2, s3, isem):
        c = lax.axis_index("c")
        s = lax.axis_index("s")
        lo = c * NLOC
        ssb = (ssb0, ssb1)
        dloc = (dloc0, dloc1)
        rows = (r0, r1, r2, r3)
        gsem = (g0, g1, g2s, g3)
        ssem = (s0, s1, s2, s3)

        def run_set(g2_hbm, src_hbm, dst_hbm, out_hbm):
            def run_pass(grp, _):
                def zrow(r, _):
                    for kk in range(CG // 16):
                        rows[0][r, pl.ds(kk * 16, 16)] = jnp.zeros(
                            (16,), jnp.float32)
                    return 0
                lax.fori_loop(0, BB, zrow, 0)
                done = 0
                while done + BB <= zrt:
                    pltpu.sync_copy(rows[0],
                                    acc.at[pl.ds(s * zrt + done, BB)])
                    done += BB
                if done < zrt:
                    pltpu.sync_copy(rows[0].at[pl.ds(0, zrt - done)],
                                    acc.at[pl.ds(s * zrt + done, zrt - done)])
                plsc.subcore_barrier()

                def load_start(si, u):
                    base = s * ept + si * SBE
                    pltpu.async_copy(src_hbm.at[pl.ds(base, SBE)], ssb[u], isem)
                    pltpu.async_copy(dst_hbm.at[pl.ds(base, SBE)], dtmp, isem)

                def load_finish(si, u):
                    base = s * ept + si * SBE
                    pltpu.make_async_copy(
                        src_hbm.at[pl.ds(base, SBE)], ssb[u], isem).wait()
                    pltpu.make_async_copy(
                        dst_hbm.at[pl.ds(base, SBE)], dtmp, isem).wait()

                    def tf(r, _):
                        for kk in range(BB // 16):
                            sl = pl.ds(r * BB + kk * 16, 16)
                            if split > 1:
                                ssb[u][sl] = ssb[u][sl] * split + grp
                            d = dtmp[sl] - lo
                            ok = (d >= 0) & (d < NLOC)
                            dloc[u][r, pl.ds(kk * 16, 16)] = jnp.where(
                                ok, d, NLOC + (dtmp[sl] & (CG - 1)))
                        return 0
                    lax.fori_loop(0, SBB, tf, 0)

                def gidx(u, jj):
                    return ssb[u].at[pl.ds(jj * BB, BB)]

                def gstart(u, jj, p):
                    pltpu.async_copy(g2_hbm.at[gidx(u, jj)], rows[p], gsem[p])

                def gwait(u, jj, p):
                    pltpu.make_async_copy(
                        g2_hbm.at[gidx(u, jj)], rows[p], gsem[p]).wait()

                def sstart(u, jj, p):
                    pltpu.async_copy(rows[p], acc.at[dloc[u].at[jj]],
                                     ssem[p], add=True)

                def swait(p):
                    pltpu.make_async_copy(rows[p], acc.at[dloc[0].at[0]],
                                          ssem[p]).wait()

                def sb_steps(si, u, first):
                    for jj in range(SBB):
                        p = jj % R
                        ph = (jj + L) % R
                        gwait(u, jj, p)
                        sstart(u, jj, p)
                        if not (first and jj < L):
                            swait(ph)
                        if jj + L < SBB:
                            gstart(u, jj + L, ph)
                        else:
                            @pl.when(si < NSB - 1)
                            def _():
                                gstart(u ^ 1, jj + L - SBB, ph)
                        if jj == 0:
                            @pl.when(si < NSB - 1)
                            def _():
                                load_start(si + 1, u ^ 1)
                        if jj == L + 1:
                            @pl.when(si < NSB - 1)
                            def _():
                                load_finish(si + 1, u ^ 1)

                load_start(0, 0)
                load_finish(0, 0)
                for jj in range(L):
                    gstart(0, jj, jj % R)
                sb_steps(0, 0, True)

                def pairfn(k2, _):
                    si = 1 + 2 * k2
                    sb_steps(si, 1, False)
                    sb_steps(si + 1, 0, False)
                    return 0
                lax.fori_loop(0, (NSB - 1) // 2, pairfn, 0)

                for jj in range(SBB - L, SBB):
                    swait(jj % R)

                if tail:
                    base = s * ept + NSB * SBE
                    pltpu.sync_copy(src_hbm.at[pl.ds(base, tail)],
                                    ssb[0].at[pl.ds(0, tail)])
                    pltpu.sync_copy(dst_hbm.at[pl.ds(base, tail)],
                                    dtmp.at[pl.ds(0, tail)])
                    for kk in range(tail // 16):
                        sl = pl.ds(kk * 16, 16)
                        if split > 1:
                            ssb[0][sl] = ssb[0][sl] * split + grp
                        d = dtmp[sl] - lo
                        ok = (d >= 0) & (d < NLOC)
                        dvt[sl] = jnp.where(ok, d, NLOC + (dtmp[sl] & (CG - 1)))
                    pltpu.async_copy(
                        g2_hbm.at[ssb[0].at[pl.ds(0, tail)]],
                        rows[0].at[pl.ds(0, tail)], gsem[0]).wait()
                    pltpu.sync_copy(rows[0].at[pl.ds(0, tail)],
                                    acc.at[dvt], add=True)
                plsc.subcore_barrier()

                for t in range(rpt // FB):
                    pltpu.sync_copy(acc.at[pl.ds(s * rpt + t * FB, FB)], fb)
                    pltpu.sync_copy(
                        fb, out_hbm.at[grp, pl.ds(lo + s * rpt + t * FB, FB)])
                plsc.subcore_barrier()
                return 0
            lax.fori_loop(0, split, run_pass, 0)

        run_set(g2q_hbm, srcq_hbm, dstq_hbm, outq_hbm)
        run_set(g2p_hbm, srcp_hbm, dstp_hbm, outp_hbm)

    return k


def _sc_conv(gq, gp, srcq, dstq, srcp, dstp, NPAD):
    N, F = gq.shape
    split = F // CG
    k = _sc_conv_kernel(srcq.shape[0], NPAD, split)
    return k(gq.reshape(split * N, CG), srcq, dstq,
             gp.reshape(split * N, CG), srcp, dstp)


@functools.lru_cache(maxsize=None)
def _tc_prep_kernel(N, F_IN, HID, NPAD):
    def body(x_ref, wq_ref, wp_ref, degq_ref, degp_ref, gq_ref, gp_ref):
        x = x_ref[...]
        mu = jnp.mean(x, axis=0, keepdims=True)
        xc = x - mu
        var = jnp.sum(xc * xc, axis=0, keepdims=True) / (N - 1)
        xs = xc * lax.rsqrt(var)
        dq = lax.rsqrt(degq_ref[...][:N])
        dp = lax.rsqrt(degp_ref[...][:N])
        gq_ref[...] = dq * jnp.dot(xs, wq_ref[...],
                                   preferred_element_type=jnp.float32)
        gp_ref[...] = dp * jnp.dot(xs, wp_ref[...],
                                   preferred_element_type=jnp.float32)

    return pl.pallas_call(
        body,
        out_shape=[jax.ShapeDtypeStruct((N, HID), jnp.float32),
                   jax.ShapeDtypeStruct((N, HID), jnp.float32)],
    )


@functools.lru_cache(maxsize=None)
def _tc_mid_kernel(N, F, FOUT, NPAD, TN, pad_out):
    NT = N // TN
    split = F // CG

    def body(accq_ref, accp_ref, gq_ref, gp_ref, degq_ref, degp_ref,
             bq_ref, bp_ref, wq_ref, wp_ref, gqn_ref, gpn_ref):
        dq = lax.rsqrt(degq_ref[...])
        dp = lax.rsqrt(degp_ref[...])
        accq = jnp.concatenate([accq_ref[g] for g in range(split)], axis=1)
        accp = jnp.concatenate([accp_ref[g] for g in range(split)], axis=1)
        outq = dq * accq + dq * gq_ref[...] + bq_ref[...]
        outp = dp * accp + dp * gp_ref[...] + bp_ref[...]
        x = ALPHA_Q * jnp.maximum(outq, 0.0) + ALPHA_P * jnp.maximum(outp, 0.0)
        hq = jnp.dot(x, wq_ref[...], preferred_element_type=jnp.float32)
        hp = jnp.dot(x, wp_ref[...], preferred_element_type=jnp.float32)
        if pad_out:
            z = jnp.zeros((TN, CG - FOUT), jnp.float32)
            gqn_ref[...] = jnp.concatenate([dq * hq, z], axis=1)
            gpn_ref[...] = jnp.concatenate([dp * hp, z], axis=1)
        else:
            gqn_ref[...] = dq * hq
            gpn_ref[...] = dp * hp

    fo = CG if pad_out else FOUT
    acc_s = pl.BlockSpec((split, TN, CG), lambda i: (0, i, 0))
    row = pl.BlockSpec((TN, F), lambda i: (i, 0))
    col = pl.BlockSpec((TN, 1), lambda i: (i, 0))
    full_b = pl.BlockSpec((1, F), lambda i: (0, 0))
    full_w = pl.BlockSpec((F, FOUT), lambda i: (0, 0))
    out_row = pl.BlockSpec((TN, fo), lambda i: (i, 0))
    return pl.pallas_call(
        body,
        grid=(NT,),
        in_specs=[acc_s, acc_s, row, row, col, col, full_b, full_b,
                  full_w, full_w],
        out_specs=[out_row, out_row],
        out_shape=[jax.ShapeDtypeStruct((N, fo), jnp.float32),
                   jax.ShapeDtypeStruct((N, fo), jnp.float32)],
    )


@functools.lru_cache(maxsize=None)
def _tc_final_kernel(N, F, NPAD, TN):
    NT = N // TN

    def body(accq_ref, accp_ref, gq_ref, gp_ref, degq_ref, degp_ref,
             bq_ref, bp_ref, out_ref):
        dq = lax.rsqrt(degq_ref[...])
        dp = lax.rsqrt(degp_ref[...])
        accq = accq_ref[0][:, :F]
        accp = accp_ref[0][:, :F]
        gq = gq_ref[...][:, :F]
        gp = gp_ref[...][:, :F]
        outq = dq * accq + dq * gq + bq_ref[...]
        outp = dp * accp + dp * gp + bp_ref[...]
        z = ALPHA_Q * outq + ALPHA_P * outp
        m = jnp.max(z, axis=1, keepdims=True)
        zs = z - m
        lse = jnp.log(jnp.sum(jnp.exp(zs), axis=1, keepdims=True))
        out_ref[...] = zs - lse

    acc_s = pl.BlockSpec((1, TN, CG), lambda i: (0, i, 0))
    row_p = pl.BlockSpec((TN, CG), lambda i: (i, 0))
    col = pl.BlockSpec((TN, 1), lambda i: (i, 0))
    full_b = pl.BlockSpec((1, F), lambda i: (0, 0))
    out_row = pl.BlockSpec((TN, F), lambda i: (i, 0))
    return pl.pallas_call(
        body,
        grid=(NT,),
        in_specs=[acc_s, acc_s, row_p, row_p, col, col, full_b, full_b],
        out_specs=out_row,
        out_shape=jax.ShapeDtypeStruct((N, F), jnp.float32),
    )


def kernel(x, edge_index_q, edge_index_p, Wq0, bq0, Wp0, bp0,
           Wq1, bq1, Wp1, bp1, Wq2, bq2, Wp2, bp2):
    N, F_IN = x.shape
    HID = Wq0.shape[1]
    NLAB = Wq2.shape[1]
    E = edge_index_q.shape[1]
    NPAD = 10240
    TN = 1000

    srcq, dstq = edge_index_q[0], edge_index_q[1]
    srcp, dstp = edge_index_p[0], edge_index_p[1]

    degq, degp = _tc_degree_kernel(E, NPAD)(dstq.reshape(E, 1),
                                            dstp.reshape(E, 1))
    degq = degq.reshape(NPAD, 1)
    degp = degp.reshape(NPAD, 1)

    gq, gp = _tc_prep_kernel(N, F_IN, HID, NPAD)(x, Wq0, Wp0, degq, degp)

    mid = _tc_mid_kernel(N, HID, HID, NPAD, TN, False)
    last = _tc_mid_kernel(N, HID, NLAB, NPAD, TN, True)

    accq, accp = _sc_conv(gq, gp, srcq, dstq, srcp, dstp, NPAD)
    gq, gp = mid(accq, accp, gq, gp, degq, degp,
                 bq0.reshape(1, HID), bp0.reshape(1, HID), Wq1, Wp1)

    accq, accp = _sc_conv(gq, gp, srcq, dstq, srcp, dstp, NPAD)
    gq, gp = mid(accq, accp, gq, gp, degq, degp,
                 bq1.reshape(1, HID), bp1.reshape(1, HID), Wq1, Wp1)

    accq, accp = _sc_conv(gq, gp, srcq, dstq, srcp, dstp, NPAD)
    gq, gp = last(accq, accp, gq, gp, degq, degp,
                  bq1.reshape(1, HID), bp1.reshape(1, HID), Wq2, Wp2)

    accq, accp = _sc_conv(gq, gp, srcq, dstq, srcp, dstp, NPAD)
    out = _tc_final_kernel(N, NLAB, NPAD, TN)(
        accq, accp, gq, gp, degq, degp,
        bq2.reshape(1, NLAB), bp2.reshape(1, NLAB))
    return out
